# baseline (speedup 1.0000x reference)
import functools

import jax
import jax.numpy as jnp
from jax import lax
from jax.experimental import pallas as pl
from jax.experimental.pallas import tpu as pltpu
from jax.experimental.pallas import tpu_sc as plsc

N_CAT = 26
VOCAB = 100000
EMBED = 32
BATCH = 16384
DENSE_DIM = 13

NC = 2
NS = 16
NW = NC * NS
LPW = (N_CAT * EMBED) // NW


def _dense_body(w_ref, d_ref, b_ref, o_ref):
    acc = lax.dot_general(w_ref[...], d_ref[...], (((0,), (0,)), ((), ())),
                          preferred_element_type=jnp.float32)
    o_ref[...] = jnp.maximum(acc + b_ref[...], 0.0)


def _dense_tc(dense_t, W, b):
    BN = 2048
    return pl.pallas_call(
        _dense_body,
        grid=(BATCH // BN,),
        in_specs=[
            pl.BlockSpec((DENSE_DIM, EMBED), lambda i: (0, 0)),
            pl.BlockSpec((DENSE_DIM, BN), lambda i: (0, i)),
            pl.BlockSpec((EMBED, 1), lambda i: (0, 0)),
        ],
        out_specs=pl.BlockSpec((EMBED, BN), lambda i: (0, i)),
        out_shape=jax.ShapeDtypeStruct((EMBED, BATCH), jnp.float32),
    )(W, dense_t, b.reshape(EMBED, 1))


def _sc_stream(cats_flat, tables_t, dlines_i):
    mesh = plsc.VectorSubcoreMesh(core_axis_name="c", subcore_axis_name="s")

    @functools.partial(
        pl.kernel,
        mesh=mesh,
        out_type=jax.ShapeDtypeStruct((N_CAT + 1, EMBED, BATCH), jnp.int32),
        scratch_types=[
            pltpu.VMEM((VOCAB,), jnp.float32),
            pltpu.VMEM((BATCH,), jnp.int32),
            pltpu.SemaphoreType.DMA,
            pltpu.SemaphoreType.DMA,
            pltpu.SemaphoreType.DMA,
        ],
        compiler_params=pltpu.CompilerParams(use_tc_tiling_on_sc=True,
                                             needs_layout_passes=False),
    )
    def k(cats_hbm, tab_hbm, dl_hbm, out_hbm, row_v, line_v, semr, semo, sem):
        wid = lax.axis_index("s") * NC + lax.axis_index("c")

        pltpu.sync_copy(dl_hbm.at[wid], line_v)
        pltpu.sync_copy(line_v, out_hbm.at[N_CAT, wid])

        def te(li):
            line = wid * LPW + li
            t = line // EMBED
            return t, line - t * EMBED

        UNROLL = 8

        def scan():
            def g(j, c2):
                for u in range(UNROLL):
                    sl = pl.ds((j * UNROLL + u) * 16, 16)
                    val = plsc.load_gather(row_v, [line_v[sl]])
                    line_v[sl] = plsc.bitcast(val, jnp.int32)
                return c2
            lax.fori_loop(0, BATCH // (16 * UNROLL), g, 0)

        t0, e0 = te(0)
        row_cp = pltpu.make_async_copy(tab_hbm.at[t0, e0, :], row_v, semr)
        row_cp.start()
        out_cp = None
        for li in range(LPW):
            t, e = te(li)
            if out_cp is not None:
                out_cp.wait()
            pltpu.sync_copy(cats_hbm.at[pl.ds(t * BATCH, BATCH)], line_v)
            row_cp.wait()
            scan()
            if li + 1 < LPW:
                t2, e2 = te(li + 1)
                row_cp = pltpu.make_async_copy(tab_hbm.at[t2, e2, :], row_v,
                                               semr)
                row_cp.start()
            out_cp = pltpu.make_async_copy(line_v, out_hbm.at[t, e, :], semo)
            out_cp.start()
        out_cp.wait()

    return k(cats_flat, tables_t, dlines_i)


def kernel(cat_0, cat_1, cat_2, cat_3, cat_4, cat_5, cat_6, cat_7, cat_8,
           cat_9, cat_10, cat_11, cat_12, cat_13, cat_14, cat_15, cat_16,
           cat_17, cat_18, cat_19, cat_20, cat_21, cat_22, cat_23, cat_24,
           cat_25, dense_0, tables, W, b):
    cats_flat = jnp.concatenate([
        cat_0, cat_1, cat_2, cat_3, cat_4, cat_5, cat_6, cat_7, cat_8,
        cat_9, cat_10, cat_11, cat_12, cat_13, cat_14, cat_15, cat_16,
        cat_17, cat_18, cat_19, cat_20, cat_21, cat_22, cat_23, cat_24,
        cat_25])
    tables_t = jnp.transpose(tables, (0, 2, 1))
    dense_t = jnp.transpose(dense_0, (1, 0))
    dlines = _dense_tc(dense_t, W, b)
    dlines_i = lax.bitcast_convert_type(dlines, jnp.int32)
    out_i = _sc_stream(cats_flat, tables_t, dlines_i)
    out_f = lax.bitcast_convert_type(out_i, jnp.float32)
    return jnp.transpose(out_f, (2, 0, 1))


# speedup vs baseline: 1.2816x; 1.2816x over previous
import functools

import jax
import jax.numpy as jnp
from jax import lax
from jax.experimental import pallas as pl
from jax.experimental.pallas import tpu as pltpu
from jax.experimental.pallas import tpu_sc as plsc

N_CAT = 26
VOCAB = 100000
EMBED = 32
BATCH = 16384
DENSE_DIM = 13

NC = 2
NS = 16
NW = NC * NS
LPW = (N_CAT * EMBED) // NW


def _dense_body(w_ref, d_ref, b_ref, o_ref):
    acc = lax.dot_general(w_ref[...], d_ref[...], (((0,), (0,)), ((), ())),
                          preferred_element_type=jnp.float32)
    o_ref[...] = jnp.maximum(acc + b_ref[...], 0.0)


def _dense_tc(dense_t, W, b):
    BN = 2048
    return pl.pallas_call(
        _dense_body,
        grid=(BATCH // BN,),
        in_specs=[
            pl.BlockSpec((DENSE_DIM, EMBED), lambda i: (0, 0)),
            pl.BlockSpec((DENSE_DIM, BN), lambda i: (0, i)),
            pl.BlockSpec((EMBED, 1), lambda i: (0, 0)),
        ],
        out_specs=pl.BlockSpec((EMBED, BN), lambda i: (0, i)),
        out_shape=jax.ShapeDtypeStruct((EMBED, BATCH), jnp.float32),
    )(W, dense_t, b.reshape(EMBED, 1))


def _sc_stream(cats_flat, tables_t, dlines_i):
    mesh = plsc.VectorSubcoreMesh(core_axis_name="c", subcore_axis_name="s")

    @functools.partial(
        pl.kernel,
        mesh=mesh,
        out_type=jax.ShapeDtypeStruct((N_CAT + 1, EMBED, BATCH), jnp.int32),
        scratch_types=[
            pltpu.VMEM((VOCAB,), jnp.float32),
            pltpu.VMEM((BATCH,), jnp.int32),
            pltpu.SemaphoreType.DMA,
            pltpu.SemaphoreType.DMA,
            pltpu.SemaphoreType.DMA,
        ],
        compiler_params=pltpu.CompilerParams(use_tc_tiling_on_sc=True,
                                             needs_layout_passes=False),
    )
    def k(cats_hbm, tab_hbm, dl_hbm, out_hbm, row_v, line_v, semr, semo, sem):
        wid = lax.axis_index("s") * NC + lax.axis_index("c")

        pltpu.sync_copy(dl_hbm.at[wid], line_v)
        pltpu.sync_copy(line_v, out_hbm.at[N_CAT, wid])

        def te(li):
            line = wid * LPW + li
            t = line // EMBED
            return t, line - t * EMBED

        UNROLL = 16

        def scan():
            def g(j, c2):
                sls = [pl.ds((j * UNROLL + u) * 16, 16) for u in range(UNROLL)]
                idxs = [line_v[sl] for sl in sls]
                vals = [plsc.load_gather(row_v, [ix]) for ix in idxs]
                for sl, val in zip(sls, vals):
                    line_v[sl] = plsc.bitcast(val, jnp.int32)
                return c2
            lax.fori_loop(0, BATCH // (16 * UNROLL), g, 0)

        t0, e0 = te(0)
        row_cp = pltpu.make_async_copy(tab_hbm.at[t0, e0, :], row_v, semr)
        row_cp.start()
        out_cp = None
        for li in range(LPW):
            t, e = te(li)
            if out_cp is not None:
                out_cp.wait()
            pltpu.sync_copy(cats_hbm.at[pl.ds(t * BATCH, BATCH)], line_v)
            row_cp.wait()
            scan()
            if li + 1 < LPW:
                t2, e2 = te(li + 1)
                row_cp = pltpu.make_async_copy(tab_hbm.at[t2, e2, :], row_v,
                                               semr)
                row_cp.start()
            out_cp = pltpu.make_async_copy(line_v, out_hbm.at[t, e, :], semo)
            out_cp.start()
        out_cp.wait()

    return k(cats_flat, tables_t, dlines_i)


def kernel(cat_0, cat_1, cat_2, cat_3, cat_4, cat_5, cat_6, cat_7, cat_8,
           cat_9, cat_10, cat_11, cat_12, cat_13, cat_14, cat_15, cat_16,
           cat_17, cat_18, cat_19, cat_20, cat_21, cat_22, cat_23, cat_24,
           cat_25, dense_0, tables, W, b):
    cats_flat = jnp.concatenate([
        cat_0, cat_1, cat_2, cat_3, cat_4, cat_5, cat_6, cat_7, cat_8,
        cat_9, cat_10, cat_11, cat_12, cat_13, cat_14, cat_15, cat_16,
        cat_17, cat_18, cat_19, cat_20, cat_21, cat_22, cat_23, cat_24,
        cat_25])
    tables_t = jnp.transpose(tables, (0, 2, 1))
    dense_t = jnp.transpose(dense_0, (1, 0))
    dlines = _dense_tc(dense_t, W, b)
    dlines_i = lax.bitcast_convert_type(dlines, jnp.int32)
    out_i = _sc_stream(cats_flat, tables_t, dlines_i)
    out_f = lax.bitcast_convert_type(out_i, jnp.float32)
    return jnp.transpose(out_f, (2, 0, 1))


# speedup vs baseline: 1.3700x; 1.0690x over previous
import functools

import jax
import jax.numpy as jnp
from jax import lax
from jax.experimental import pallas as pl
from jax.experimental.pallas import tpu as pltpu
from jax.experimental.pallas import tpu_sc as plsc

N_CAT = 26
VOCAB = 100000
EMBED = 32
BATCH = 16384
DENSE_DIM = 13

NC = 2
NS = 16
NW = NC * NS
LPW = (N_CAT * EMBED) // NW


def _dense_body(w_ref, d_ref, b_ref, o_ref):
    acc = lax.dot_general(w_ref[...], d_ref[...], (((0,), (0,)), ((), ())),
                          preferred_element_type=jnp.float32)
    o_ref[...] = jnp.maximum(acc + b_ref[...], 0.0)


def _dense_tc(dense_t, W, b):
    BN = 2048
    return pl.pallas_call(
        _dense_body,
        grid=(BATCH // BN,),
        in_specs=[
            pl.BlockSpec((DENSE_DIM, EMBED), lambda i: (0, 0)),
            pl.BlockSpec((DENSE_DIM, BN), lambda i: (0, i)),
            pl.BlockSpec((EMBED, 1), lambda i: (0, 0)),
        ],
        out_specs=pl.BlockSpec((EMBED, BN), lambda i: (0, i)),
        out_shape=jax.ShapeDtypeStruct((EMBED, BATCH), jnp.float32),
    )(W, dense_t, b.reshape(EMBED, 1))


def _sc_stream(cats_flat, tables_t, dlines_i):
    mesh = plsc.VectorSubcoreMesh(core_axis_name="c", subcore_axis_name="s")

    @functools.partial(
        pl.kernel,
        mesh=mesh,
        out_type=jax.ShapeDtypeStruct((N_CAT + 1, EMBED, BATCH), jnp.int32),
        scratch_types=[
            pltpu.VMEM((VOCAB,), jnp.float32),
            pltpu.VMEM((BATCH,), jnp.int32),
            pltpu.VMEM((14336,), jnp.int32),
            pltpu.SemaphoreType.DMA,
            pltpu.SemaphoreType.DMA,
            pltpu.SemaphoreType.DMA,
        ],
        compiler_params=pltpu.CompilerParams(use_tc_tiling_on_sc=True,
                                             needs_layout_passes=False),
    )
    def k(cats_hbm, tab_hbm, dl_hbm, out_hbm, row_v, line_v, cache_v,
          semr, semo, sem):
        wid = lax.axis_index("s") * NC + lax.axis_index("c")

        pltpu.sync_copy(dl_hbm.at[wid], line_v)
        pltpu.sync_copy(line_v, out_hbm.at[N_CAT, wid])

        def te(li):
            line = wid * LPW + li
            t = line // EMBED
            return t, line - t * EMBED

        UNROLL = 16
        HEAD = 14336

        def seg(src_v, n_groups, base):
            def g(j, c2):
                sls = [pl.ds(base + (j * UNROLL + u) * 16, 16)
                       for u in range(UNROLL)]
                idxs = [src_v[sl] for sl in sls]
                vals = [plsc.load_gather(row_v, [ix]) for ix in idxs]
                for sl2, val in zip(sls, vals):
                    line_v[sl2] = plsc.bitcast(val, jnp.int32)
                return c2
            lax.fori_loop(0, n_groups // UNROLL, g, 0)

        t0, e0 = te(0)
        row_cp = pltpu.make_async_copy(tab_hbm.at[t0, e0, :], row_v, semr)
        row_cp.start()
        prev_t = jnp.int32(-1)
        out_cps = []
        for li in range(LPW):
            t, e = te(li)
            for cp in out_cps:
                cp.wait()

            @pl.when(t != prev_t)
            def _():
                pltpu.sync_copy(cats_hbm.at[pl.ds(t * BATCH, HEAD)], cache_v)

            prev_t = t
            pltpu.sync_copy(
                cats_hbm.at[pl.ds(t * BATCH + HEAD, BATCH - HEAD)],
                line_v.at[pl.ds(HEAD, BATCH - HEAD)])
            row_cp.wait()
            seg(cache_v, HEAD // 16, 0)
            cp1 = pltpu.make_async_copy(line_v.at[pl.ds(0, HEAD)],
                                        out_hbm.at[t, e, pl.ds(0, HEAD)],
                                        semo)
            cp1.start()
            seg(line_v, (BATCH - HEAD) // 16, HEAD)
            if li + 1 < LPW:
                t2, e2 = te(li + 1)
                row_cp = pltpu.make_async_copy(tab_hbm.at[t2, e2, :], row_v,
                                               semr)
                row_cp.start()
            cp2 = pltpu.make_async_copy(
                line_v.at[pl.ds(HEAD, BATCH - HEAD)],
                out_hbm.at[t, e, pl.ds(HEAD, BATCH - HEAD)], semo)
            cp2.start()
            out_cps = [cp1, cp2]
        for cp in out_cps:
            cp.wait()

    return k(cats_flat, tables_t, dlines_i)


def kernel(cat_0, cat_1, cat_2, cat_3, cat_4, cat_5, cat_6, cat_7, cat_8,
           cat_9, cat_10, cat_11, cat_12, cat_13, cat_14, cat_15, cat_16,
           cat_17, cat_18, cat_19, cat_20, cat_21, cat_22, cat_23, cat_24,
           cat_25, dense_0, tables, W, b):
    cats_flat = jnp.concatenate([
        cat_0, cat_1, cat_2, cat_3, cat_4, cat_5, cat_6, cat_7, cat_8,
        cat_9, cat_10, cat_11, cat_12, cat_13, cat_14, cat_15, cat_16,
        cat_17, cat_18, cat_19, cat_20, cat_21, cat_22, cat_23, cat_24,
        cat_25])
    tables_t = jnp.transpose(tables, (0, 2, 1))
    dense_t = jnp.transpose(dense_0, (1, 0))
    dlines = _dense_tc(dense_t, W, b)
    dlines_i = lax.bitcast_convert_type(dlines, jnp.int32)
    out_i = _sc_stream(cats_flat, tables_t, dlines_i)
    out_f = lax.bitcast_convert_type(out_i, jnp.float32)
    return jnp.transpose(out_f, (2, 0, 1))


# speedup vs baseline: 1.3894x; 1.0141x over previous
import functools

import jax
import jax.numpy as jnp
from jax import lax
from jax.experimental import pallas as pl
from jax.experimental.pallas import tpu as pltpu
from jax.experimental.pallas import tpu_sc as plsc

N_CAT = 26
VOCAB = 100000
EMBED = 32
BATCH = 16384
DENSE_DIM = 13

NC = 2
NS = 16
NW = NC * NS
LPW = (N_CAT * EMBED) // NW


def _dense_body(w_ref, d_ref, b_ref, o_ref):
    acc = lax.dot_general(w_ref[...], d_ref[...], (((0,), (0,)), ((), ())),
                          preferred_element_type=jnp.float32)
    o_ref[...] = jnp.maximum(acc + b_ref[...], 0.0)


def _dense_tc(dense_t, W, b):
    BN = 2048
    return pl.pallas_call(
        _dense_body,
        grid=(BATCH // BN,),
        in_specs=[
            pl.BlockSpec((DENSE_DIM, EMBED), lambda i: (0, 0)),
            pl.BlockSpec((DENSE_DIM, BN), lambda i: (0, i)),
            pl.BlockSpec((EMBED, 1), lambda i: (0, 0)),
        ],
        out_specs=pl.BlockSpec((EMBED, BN), lambda i: (0, i)),
        out_shape=jax.ShapeDtypeStruct((EMBED, BATCH), jnp.float32),
    )(W, dense_t, b.reshape(EMBED, 1))


def _sc_stream(cats_flat, tables_t, dlines_i):
    mesh = plsc.VectorSubcoreMesh(core_axis_name="c", subcore_axis_name="s")

    @functools.partial(
        pl.kernel,
        mesh=mesh,
        out_type=jax.ShapeDtypeStruct((N_CAT + 1, EMBED, BATCH), jnp.int32),
        scratch_types=[
            pltpu.VMEM((VOCAB,), jnp.float32),
            pltpu.VMEM((BATCH,), jnp.int32),
            pltpu.VMEM((14336,), jnp.int32),
            pltpu.SemaphoreType.DMA,
            pltpu.SemaphoreType.DMA,
            pltpu.SemaphoreType.DMA,
        ],
        compiler_params=pltpu.CompilerParams(use_tc_tiling_on_sc=True,
                                             needs_layout_passes=False),
    )
    def k(cats_hbm, tab_hbm, dl_hbm, out_hbm, row_v, line_v, cache_v,
          semr, semo, sem):
        wid = lax.axis_index("s") * NC + lax.axis_index("c")

        def te(li):
            line = wid * LPW + li
            t = line // EMBED
            return t, line - t * EMBED

        UNROLL = 16
        HEAD = 14336

        def seg(src_v, n_groups, base):
            def g(j, c2):
                sls = [pl.ds(base + (j * UNROLL + u) * 16, 16)
                       for u in range(UNROLL)]
                idxs = [src_v[sl] for sl in sls]
                vals = [plsc.load_gather(row_v, [ix]) for ix in idxs]
                for sl2, val in zip(sls, vals):
                    line_v[sl2] = plsc.bitcast(val, jnp.int32)
                return c2
            lax.fori_loop(0, n_groups // UNROLL, g, 0)

        t0, e0 = te(0)
        row_cp = pltpu.make_async_copy(tab_hbm.at[t0, e0, :], row_v, semr)
        row_cp.start()
        pltpu.sync_copy(dl_hbm.at[wid], line_v)
        pltpu.sync_copy(line_v, out_hbm.at[N_CAT, wid])
        prev_t = jnp.int32(-1)
        out_cps = []
        for li in range(LPW):
            t, e = te(li)
            for cp in out_cps:
                cp.wait()

            @pl.when(t != prev_t)
            def _():
                pltpu.sync_copy(cats_hbm.at[pl.ds(t * BATCH, HEAD)], cache_v)

            prev_t = t
            pltpu.sync_copy(
                cats_hbm.at[pl.ds(t * BATCH + HEAD, BATCH - HEAD)],
                line_v.at[pl.ds(HEAD, BATCH - HEAD)])
            row_cp.wait()
            seg(cache_v, HEAD // 16, 0)
            cp1 = pltpu.make_async_copy(line_v.at[pl.ds(0, HEAD)],
                                        out_hbm.at[t, e, pl.ds(0, HEAD)],
                                        semo)
            cp1.start()
            seg(line_v, (BATCH - HEAD) // 16, HEAD)
            if li + 1 < LPW:
                t2, e2 = te(li + 1)
                row_cp = pltpu.make_async_copy(tab_hbm.at[t2, e2, :], row_v,
                                               semr)
                row_cp.start()
            cp2 = pltpu.make_async_copy(
                line_v.at[pl.ds(HEAD, BATCH - HEAD)],
                out_hbm.at[t, e, pl.ds(HEAD, BATCH - HEAD)], semo)
            cp2.start()
            out_cps = [cp1, cp2]
        for cp in out_cps:
            cp.wait()

    return k(cats_flat, tables_t, dlines_i)


def kernel(cat_0, cat_1, cat_2, cat_3, cat_4, cat_5, cat_6, cat_7, cat_8,
           cat_9, cat_10, cat_11, cat_12, cat_13, cat_14, cat_15, cat_16,
           cat_17, cat_18, cat_19, cat_20, cat_21, cat_22, cat_23, cat_24,
           cat_25, dense_0, tables, W, b):
    cats_flat = jnp.concatenate([
        cat_0, cat_1, cat_2, cat_3, cat_4, cat_5, cat_6, cat_7, cat_8,
        cat_9, cat_10, cat_11, cat_12, cat_13, cat_14, cat_15, cat_16,
        cat_17, cat_18, cat_19, cat_20, cat_21, cat_22, cat_23, cat_24,
        cat_25])
    tables_t = jnp.transpose(tables, (0, 2, 1))
    dense_t = jnp.transpose(dense_0, (1, 0))
    dlines = _dense_tc(dense_t, W, b)
    dlines_i = lax.bitcast_convert_type(dlines, jnp.int32)
    out_i = _sc_stream(cats_flat, tables_t, dlines_i)
    out_f = lax.bitcast_convert_type(out_i, jnp.float32)
    return jnp.transpose(out_f, (2, 0, 1))
